# Initial kernel scaffold; baseline (speedup 1.0000x reference)
import jax
import jax.numpy as jnp
from jax.experimental import pallas as pl


def kernel(words, table):
    raise NotImplementedError("write your pallas kernel here")


# speedup vs baseline: 4.0940x; 4.0940x over previous
import functools

import jax
import jax.numpy as jnp
from jax import lax
from jax.experimental import pallas as pl
from jax.experimental.pallas import tpu as pltpu
from jax.experimental.pallas import tpu_sc as plsc

NC = 2
NS = 16
NW = NC * NS
LANES = 16

EMBED_DIM = 64
CHUNK = 128


def _body(idx_hbm, table_hbm, out_hbm, idx_v, rows_v, zcnt_v, gsem):
    steps = idx_hbm.shape[1]
    wid = lax.axis_index("s") * NC + lax.axis_index("c")

    pltpu.sync_copy(idx_hbm.at[wid], idx_v)

    zeros16 = jnp.zeros((LANES,), jnp.float32)
    rowiota = lax.iota(jnp.int32, LANES)

    def step(j, carry):
        pltpu.async_copy(table_hbm.at[idx_v.at[j]], rows_v, gsem).wait()

        acc = jnp.zeros((LANES,), jnp.int32)
        for k in range(CHUNK // LANES):
            iv = idx_v[j, pl.ds(k * LANES, LANES)]
            acc = acc | jnp.where(iv == 0, jnp.int32(1), jnp.int32(0))
        flag = acc[0]
        for e in range(1, LANES):
            flag = flag | acc[e]
        anyz = flag > 0

        @pl.when(anyz)
        def _fixup():
            for k in range(CHUNK // LANES):
                iv = idx_v[j, pl.ds(k * LANES, LANES)]
                zm = jnp.where(iv == 0, jnp.int32(1), jnp.int32(0))
                for e in range(LANES):
                    @pl.when(zm[e] > 0)
                    def _zero_row(k=k, e=e):
                        for col in range(0, EMBED_DIM, LANES):
                            rows_v[k * LANES + e, pl.ds(col, LANES)] = zeros16

        base = (wid * steps + j) * CHUNK
        pltpu.sync_copy(rows_v, out_hbm.at[pl.ds(base, CHUNK)])
        return carry

    lax.fori_loop(0, steps, step, jnp.int32(0))


def kernel(words, table):
    B, H = words.shape
    V, D = table.shape
    n = B * H
    assert D == EMBED_DIM and n % (NW * CHUNK) == 0

    idx = words.reshape(NW, n // (NW * CHUNK), CHUNK).astype(jnp.int32)

    mesh = plsc.VectorSubcoreMesh(core_axis_name="c", subcore_axis_name="s")
    steps = n // (NW * CHUNK)

    run = functools.partial(
        pl.kernel,
        out_type=jax.ShapeDtypeStruct((n, D), jnp.float32),
        mesh=mesh,
        compiler_params=pltpu.CompilerParams(use_tc_tiling_on_sc=False),
        scratch_types=[
            pltpu.VMEM((steps, CHUNK), jnp.int32),
            pltpu.VMEM((CHUNK, D), jnp.float32),
            pltpu.VMEM((LANES,), jnp.int32),
            pltpu.SemaphoreType.DMA,
        ],
    )(_body)

    out = run(idx, table)
    return out.reshape(B, H, D)


# speedup vs baseline: 4.7065x; 1.1496x over previous
import functools

import jax
import jax.numpy as jnp
from jax import lax
from jax.experimental import pallas as pl
from jax.experimental.pallas import tpu as pltpu
from jax.experimental.pallas import tpu_sc as plsc

NC = 2
NS = 16
NW = NC * NS
LANES = 16

EMBED_DIM = 64
CHUNK = 128
NBUF = 5


def _body(idx_hbm, table_hbm, out_hbm, idx_v, rows_v, gsem, wsem):
    steps = idx_hbm.shape[1]
    nblk = steps // NBUF
    wid = lax.axis_index("s") * NC + lax.axis_index("c")

    pltpu.sync_copy(idx_hbm.at[wid], idx_v)

    zeros16 = jnp.zeros((LANES,), jnp.float32)

    def scan_row(r, acc):
        for k in range(CHUNK // LANES):
            iv = idx_v[r, pl.ds(k * LANES, LANES)]
            acc = acc | jnp.where(iv == 0, jnp.int32(1), jnp.int32(0))
        return acc

    acc = lax.fori_loop(0, steps, scan_row, jnp.zeros((LANES,), jnp.int32))
    wflag = acc[0]
    for e in range(1, LANES):
        wflag = wflag | acc[e]
    worker_has_zero = wflag > 0

    def gather(j, b):
        return pltpu.make_async_copy(
            table_hbm.at[idx_v.at[j]], rows_v.at[b], gsem.at[b])

    def write(j, b):
        return pltpu.make_async_copy(
            rows_v.at[b], out_hbm.at[pl.ds((wid * steps + j) * CHUNK, CHUNK)],
            wsem.at[b])

    def fixup(j, b):
        @pl.when(worker_has_zero)
        def _check():
            cacc = jnp.zeros((LANES,), jnp.int32)
            for k in range(CHUNK // LANES):
                iv = idx_v[j, pl.ds(k * LANES, LANES)]
                cacc = cacc | jnp.where(iv == 0, jnp.int32(1), jnp.int32(0))
            flag = cacc[0]
            for e in range(1, LANES):
                flag = flag | cacc[e]

            @pl.when(flag > 0)
            def _do():
                def grp(k, c):
                    iv = idx_v[j, pl.ds(k * LANES, LANES)]
                    zm = jnp.where(iv == 0, jnp.int32(1), jnp.int32(0))
                    for e in range(LANES):
                        @pl.when(zm[e] > 0)
                        def _zero_row(k=k, e=e):
                            row = k * LANES + e
                            for col in range(0, EMBED_DIM, LANES):
                                rows_v[b, row, pl.ds(col, LANES)] = zeros16
                    return c

                lax.fori_loop(0, CHUNK // LANES, grp, jnp.int32(0))

    for b in range(NBUF):
        gather(b, b).start()

    def block(g, carry):
        for b in range(NBUF):
            j = g * NBUF + b
            gather(j, b).wait()
            fixup(j, b)
            write(j, b).start()

            @pl.when(g < nblk - 1)
            def _next(j=j, b=b):
                write(j, b).wait()
                gather(j + NBUF, b).start()
        return carry

    lax.fori_loop(0, nblk, block, jnp.int32(0))

    for b in range(NBUF):
        write((nblk - 1) * NBUF + b, b).wait()


def kernel(words, table):
    B, H = words.shape
    V, D = table.shape
    n = B * H
    steps = n // (NW * CHUNK)
    assert D == EMBED_DIM and n % (NW * CHUNK) == 0 and steps % NBUF == 0

    idx = words.reshape(NW, steps, CHUNK).astype(jnp.int32)

    mesh = plsc.VectorSubcoreMesh(core_axis_name="c", subcore_axis_name="s")

    run = functools.partial(
        pl.kernel,
        out_type=jax.ShapeDtypeStruct((n, D), jnp.float32),
        mesh=mesh,
        compiler_params=pltpu.CompilerParams(use_tc_tiling_on_sc=False),
        scratch_types=[
            pltpu.VMEM((steps, CHUNK), jnp.int32),
            pltpu.VMEM((NBUF, CHUNK, EMBED_DIM), jnp.float32),
            pltpu.SemaphoreType.DMA((NBUF,)),
            pltpu.SemaphoreType.DMA((NBUF,)),
        ],
    )(_body)

    out = run(idx, table)
    return out.reshape(B, H, D)
